# baseline (speedup 1.0000x reference)
import functools

import jax
import jax.numpy as jnp
from jax import lax
from jax.experimental import pallas as pl
from jax.experimental.pallas import tpu as pltpu
from jax.experimental.pallas import tpu_sc as plsc

NC, NS, L = 2, 16, 16
NW = NC * NS
B = 64
N = 512
TE = N * (N + 1) // 2
NT = B * TE
RB = 32
NBLK = N // RB
BPW = B // NW
CH = RB * N
BUFW = CH + 16
TBUF = BUFW + 16


def _sc_body(inp_hbm, tgt_hbm, out_hbm, pbuf, tbuf, obuf):
    wid = lax.axis_index("s") * NC + lax.axis_index("c")
    zero = jnp.zeros((L,), jnp.float32)
    lanes = lax.iota(jnp.int32, L)

    def block(i, carry):
        acc0, cnt0 = carry
        b = wid * BPW + i // NBLK
        rb = (i % NBLK) * RB
        pstart = pl.multiple_of(b * (N * N) + rb * N, N)
        pltpu.sync_copy(inp_hbm.at[pl.ds(pstart, CH)], pbuf)
        off_rb = b * TE + rb * N - (rb * (rb - 1)) // 2
        s = pl.multiple_of(jnp.minimum(off_rb & ~15, NT - BUFW), 16)
        pltpu.sync_copy(tgt_hbm.at[pl.ds(s, BUFW)], tbuf.at[pl.ds(16, BUFW)])
        delta = off_rb - s

        def row(rl, rcarry):
            acc1, cnt1, seg = rcarry
            rg = rb + rl
            base_p = rl * N
            base_t = seg - rg
            j0 = rg // L

            def chunk(j, ccarry):
                acc2, cnt2 = ccarry
                c0 = j * L
                p = pbuf[pl.ds(base_p + c0, L)]
                t = tbuf[pl.ds(base_t + c0, L)]
                c = c0 + lanes
                m = (c >= rg) & (t == t)
                d = jnp.where(m, p - t, 0.0)
                return acc2 + d * d, cnt2 + jnp.where(m, 1.0, 0.0)

            acc1, cnt1 = lax.fori_loop(j0, N // L, chunk, (acc1, cnt1))
            return acc1, cnt1, seg + (N - rg)

        acc0, cnt0, _ = lax.fori_loop(0, RB, row, (acc0, cnt0, 16 + delta))
        return acc0, cnt0

    acc, cnt = lax.fori_loop(0, BPW * NBLK, block, (zero, zero))
    obuf[pl.ds(0, L)] = acc
    obuf[pl.ds(L, L)] = cnt
    pltpu.sync_copy(obuf, out_hbm.at[wid])


@jax.jit
def kernel(input, target):
    inp = input.reshape(-1)
    tgt = target.reshape(-1)
    mesh = plsc.VectorSubcoreMesh(core_axis_name="c", subcore_axis_name="s")
    run = functools.partial(
        pl.kernel,
        mesh=mesh,
        out_type=jax.ShapeDtypeStruct((NW, 2 * L), jnp.float32),
        scratch_types=[
            pltpu.VMEM((CH,), jnp.float32),
            pltpu.VMEM((TBUF,), jnp.float32),
            pltpu.VMEM((2 * L,), jnp.float32),
        ],
    )(_sc_body)
    out = run(inp, tgt)
    ssum = out[:, :L].sum()
    cnt = out[:, L:].sum()
    return jnp.where(cnt == 0, jnp.float32(0.0),
                     ssum / jnp.maximum(cnt, 1.0))


# speedup vs baseline: 1.9326x; 1.9326x over previous
import functools

import jax
import jax.numpy as jnp
from jax import lax
from jax.experimental import pallas as pl
from jax.experimental.pallas import tpu as pltpu
from jax.experimental.pallas import tpu_sc as plsc

NC, NS, L = 2, 16, 16
NW = NC * NS
B = 64
N = 512
TE = N * (N + 1) // 2
NT = B * TE
RB = 32
NBLK = N // RB
BPW = B // NW
NKB = BPW * NBLK
CH = RB * N
BUFW = CH + 16
TBUF = BUFW + 16


def _sc_body(inp_hbm, tgt_hbm, out_hbm,
             pb0, pb1, tb0, tb1, obuf, ps0, ps1, ts0, ts1):
    wid = lax.axis_index("s") * NC + lax.axis_index("c")
    lanes = lax.iota(jnp.int32, L)
    pbufs, tbufs = (pb0, pb1), (tb0, tb1)
    psems, tsems = (ps0, ps1), (ts0, ts1)

    def issue(k):
        par = k & 1
        bi, rbi = divmod(k, NBLK)
        rb = rbi * RB
        cb = rb & ~127
        W = N - cb
        b = wid * BPW + bi
        r0 = pl.multiple_of(b * N + rb, 8)
        hp = pltpu.async_copy(
            inp_hbm.at[pl.ds(r0, RB), pl.ds(cb, W)],
            pbufs[par].at[:, pl.ds(0, W)], psems[par])
        orb = rb * N - rb * (rb - 1) // 2
        sl = min(orb & ~15, TE - BUFW)
        ht = pltpu.async_copy(
            tgt_hbm.at[pl.ds(pl.multiple_of(b * TE + sl, 16), BUFW)],
            tbufs[par].at[pl.ds(16, BUFW)], tsems[par])
        return hp, ht

    def compute(k, acc, cnt):
        par = k & 1
        pb, tb = pbufs[par], tbufs[par]
        rb = (k % NBLK) * RB
        cb = rb & ~127
        orb = rb * N - rb * (rb - 1) // 2
        delta = orb - min(orb & ~15, TE - BUFW)
        jlo = rb // L

        def row(rl, carry):
            acc1, cnt1, seg = carry
            rg = rb + rl
            base_t = seg - rg

            def jbody(j, c2):
                a, cn = c2
                c0 = j * L
                p = pb[rl, pl.ds(c0 - cb, L)]
                t = tb[pl.ds(base_t + c0, L)]
                m = (lanes >= rg - c0) & (t == t)
                d = jnp.where(m, p - t, 0.0)
                return a + d * d, cn + jnp.where(m, 1.0, 0.0)

            acc1, cnt1 = lax.fori_loop(jlo, N // L, jbody, (acc1, cnt1))
            return acc1, cnt1, seg + (N - rg)

        acc, cnt, _ = lax.fori_loop(0, RB, row, (acc, cnt, 16 + delta))
        return acc, cnt

    acc = jnp.zeros((L,), jnp.float32)
    cnt = jnp.zeros((L,), jnp.float32)
    hs = [None, None]
    hs[0] = issue(0)
    for k in range(NKB):
        if k + 1 < NKB:
            hs[(k + 1) & 1] = issue(k + 1)
        hp, ht = hs[k & 1]
        hp.wait()
        ht.wait()
        acc, cnt = compute(k, acc, cnt)

    obuf[pl.ds(0, L)] = acc
    obuf[pl.ds(L, L)] = cnt
    pltpu.sync_copy(obuf, out_hbm.at[wid])


@jax.jit
def kernel(input, target):
    inp = input.reshape(B * N, N)
    tgt = target.reshape(NT)
    mesh = plsc.VectorSubcoreMesh(core_axis_name="c", subcore_axis_name="s")
    run = functools.partial(
        pl.kernel,
        mesh=mesh,
        out_type=jax.ShapeDtypeStruct((NW, 2 * L), jnp.float32),
        scratch_types=[
            pltpu.VMEM((RB, N), jnp.float32),
            pltpu.VMEM((RB, N), jnp.float32),
            pltpu.VMEM((TBUF,), jnp.float32),
            pltpu.VMEM((TBUF,), jnp.float32),
            pltpu.VMEM((2 * L,), jnp.float32),
            pltpu.SemaphoreType.DMA,
            pltpu.SemaphoreType.DMA,
            pltpu.SemaphoreType.DMA,
            pltpu.SemaphoreType.DMA,
        ],
    )(_sc_body)
    out = run(inp, tgt)
    ssum = out[:, :L].sum()
    cnt = out[:, L:].sum()
    return jnp.where(cnt == 0, jnp.float32(0.0),
                     ssum / jnp.maximum(cnt, 1.0))


# speedup vs baseline: 2.1319x; 1.1031x over previous
import functools

import jax
import jax.numpy as jnp
from jax import lax
from jax.experimental import pallas as pl
from jax.experimental.pallas import tpu as pltpu
from jax.experimental.pallas import tpu_sc as plsc

NC, NS, L = 2, 16, 16
NW = NC * NS
B = 64
N = 512
TE = N * (N + 1) // 2
RB = 8
NBLK = N // RB
TROWMAX = 4480


def _blk(k):
    rb = k * RB
    cb = rb & ~127
    W = N - cb
    lb = RB * (N - rb) - 28
    bufw = ((lb + 128 + 127) // 128) * 128
    orb = rb * N - rb * (rb - 1) // 2
    sl = min(orb & ~127, TE - bufw)
    return rb, cb, W, bufw, sl, orb - sl


def _sc_body(inp_hbm, tgt_hbm, out_hbm,
             pb0, pb1, tb0, tb1, obuf, ps0, ps1, ts0, ts1):
    wid = lax.axis_index("s") * NC + lax.axis_index("c")
    g = wid // 4
    ro = (wid % 4) * 2
    b0 = pl.multiple_of(g * 8, 8)
    lanes = lax.iota(jnp.int32, L)
    pbufs, tbufs = (pb0, pb1), (tb0, tb1)
    psems, tsems = (ps0, ps1), (ts0, ts1)

    def issue(k):
        par = k & 1
        rb, cb, W, bufw, sl, _ = _blk(k)
        hps = []
        for bb in range(2):
            r0 = pl.multiple_of((b0 + ro + bb) * N + rb, 8)
            hps.append(pltpu.async_copy(
                inp_hbm.at[pl.ds(r0, RB), pl.ds(cb, W)],
                pbufs[par].at[pl.ds(bb * RB, RB), pl.ds(0, W)],
                psems[par]))
        ht = pltpu.async_copy(
            tgt_hbm.at[pl.ds(b0, 8), pl.ds(sl, bufw)],
            tbufs[par].at[:, pl.ds(128, bufw)], tsems[par])
        return hps, ht

    def compute(k, acc, cnt):
        par = k & 1
        pb, tb = pbufs[par], tbufs[par]
        rb, cb, W, bufw, sl, delta = _blk(k)
        jlo = rb // L

        def batch_body(bb, cab):
            acc1, cnt1 = cab
            row2 = ro + bb

            def row(rl, carry):
                acc2, cnt2, seg = carry
                rg = rb + rl
                obase = seg - rg
                sh = obase & 15
                ob16 = obase - sh
                pidx = (lanes + sh) & 15
                low = lanes < (16 - sh)
                a0 = tb[row2, pl.ds(pl.multiple_of(ob16 + jlo * L, L), L)]

                def jbody(j, c2):
                    av, a, cn = c2
                    c0 = pl.multiple_of(j * L, L)
                    p = pb[bb * RB + rl, pl.ds(c0 - cb, L)]
                    bv = tb[row2, pl.ds(pl.multiple_of(ob16 + c0 + L, L), L)]
                    t = jnp.where(
                        low,
                        jnp.take_along_axis(av, pidx, 0),
                        jnp.take_along_axis(bv, pidx, 0))
                    m = (lanes >= rg - c0) & (t == t)
                    d = jnp.where(m, p - t, 0.0)
                    return bv, a + d * d, cn + jnp.where(m, 1.0, 0.0)

                _, acc2, cnt2 = lax.fori_loop(jlo, N // L, jbody,
                                              (a0, acc2, cnt2))
                return acc2, cnt2, seg + (N - rg)

            acc1, cnt1, _ = lax.fori_loop(0, RB, row,
                                          (acc1, cnt1, 128 + delta))
            return acc1, cnt1

        return lax.fori_loop(0, 2, batch_body, (acc, cnt))

    acc = jnp.zeros((L,), jnp.float32)
    cnt = jnp.zeros((L,), jnp.float32)
    hs = [None, None]
    hs[0] = issue(0)
    for k in range(NBLK):
        if k + 1 < NBLK:
            hs[(k + 1) & 1] = issue(k + 1)
        hps, ht = hs[k & 1]
        for h in hps:
            h.wait()
        ht.wait()
        acc, cnt = compute(k, acc, cnt)

    obuf[pl.ds(0, L)] = acc
    obuf[pl.ds(L, L)] = cnt
    pltpu.sync_copy(obuf, out_hbm.at[wid])


@jax.jit
def kernel(input, target):
    inp = input.reshape(B * N, N)
    mesh = plsc.VectorSubcoreMesh(core_axis_name="c", subcore_axis_name="s")
    run = functools.partial(
        pl.kernel,
        mesh=mesh,
        out_type=jax.ShapeDtypeStruct((NW, 2 * L), jnp.float32),
        scratch_types=[
            pltpu.VMEM((2 * RB, N), jnp.float32),
            pltpu.VMEM((2 * RB, N), jnp.float32),
            pltpu.VMEM((8, TROWMAX), jnp.float32),
            pltpu.VMEM((8, TROWMAX), jnp.float32),
            pltpu.VMEM((2 * L,), jnp.float32),
            pltpu.SemaphoreType.DMA,
            pltpu.SemaphoreType.DMA,
            pltpu.SemaphoreType.DMA,
            pltpu.SemaphoreType.DMA,
        ],
    )(_sc_body)
    out = run(inp, target)
    ssum = out[:, :L].sum()
    cnt = out[:, L:].sum()
    return jnp.where(cnt == 0, jnp.float32(0.0),
                     ssum / jnp.maximum(cnt, 1.0))


# speedup vs baseline: 2.2851x; 1.0718x over previous
import functools

import jax
import jax.numpy as jnp
from jax import lax
from jax.experimental import pallas as pl
from jax.experimental.pallas import tpu as pltpu
from jax.experimental.pallas import tpu_sc as plsc

NC, NS, L = 2, 16, 16
NW = NC * NS
B = 64
N = 512
TE = N * (N + 1) // 2
RB = 8
NBLK = N // RB
TROWMAX = 4480


def _blk(k):
    rb = k * RB
    cb = rb & ~127
    W = N - cb
    lb = RB * (N - rb) - 28
    bufw = ((lb + 128 + 127) // 128) * 128
    orb = rb * N - rb * (rb - 1) // 2
    sl = min(orb & ~127, TE - bufw)
    return rb, cb, W, bufw, sl, orb - sl


def _sc_body(inp_hbm, tgt_hbm, out_hbm,
             pb0, pb1, tb0, tb1, obuf, ps0, ps1, ts0, ts1):
    wid = lax.axis_index("s") * NC + lax.axis_index("c")
    g = wid // 4
    ro = (wid % 4) * 2
    b0 = pl.multiple_of(g * 8, 8)
    lanes = lax.iota(jnp.int32, L)
    pbufs, tbufs = (pb0, pb1), (tb0, tb1)
    psems, tsems = (ps0, ps1), (ts0, ts1)

    def issue(k):
        par = k & 1
        rb, cb, W, bufw, sl, _ = _blk(k)
        hps = []
        for bb in range(2):
            r0 = pl.multiple_of((b0 + ro + bb) * N + rb, 8)
            hps.append(pltpu.async_copy(
                inp_hbm.at[pl.ds(r0, RB), pl.ds(cb, W)],
                pbufs[par].at[pl.ds(bb * RB, RB), pl.ds(0, W)],
                psems[par]))
        ht = pltpu.async_copy(
            tgt_hbm.at[pl.ds(b0, 8), pl.ds(sl, bufw)],
            tbufs[par].at[:, pl.ds(128, bufw)], tsems[par])
        return hps, ht

    def compute(k, accs):
        par = k & 1
        pb, tb = pbufs[par], tbufs[par]
        rb, cb, W, bufw, sl, delta = _blk(k)
        jlo = rb // L
        nst = (N // L - 1) - jlo
        js = jlo + 1 + (nst & 1)
        npair = (N // L - js) // 2
        one = jnp.float32(1.0)
        zero = jnp.float32(0.0)

        def batch_body(bb, cab):
            aa, ca, ab2, cb2 = cab
            row2 = ro + bb

            def row(rl, carry):
                aa, ca, ab2, cb2, seg = carry
                rg = rb + rl
                obase = seg - rg
                sh = obase & 15
                ob16 = obase - sh
                pidx = (lanes + sh) & 15
                low = lanes < (16 - sh)

                def tchunk(c0, pb_prev):
                    bv = tb[row2,
                            pl.ds(pl.multiple_of(ob16 + c0 + L, L), L)]
                    pb_new = jnp.take_along_axis(bv, pidx, 0)
                    return jnp.where(low, pb_prev, pb_new), pb_new

                c0 = jlo * L
                av = tb[row2, pl.ds(pl.multiple_of(ob16 + c0, L), L)]
                pa = jnp.take_along_axis(av, pidx, 0)
                t, pprev = tchunk(c0, pa)
                p = pb[bb * RB + rl, pl.ds(c0 - cb, L)]
                m = (lanes >= rg - c0) & (t == t)
                d = jnp.where(m, p - t, 0.0)
                aa = aa + d * d
                ca = ca + jnp.where(m, one, zero)

                if nst & 1:
                    c0 = (jlo + 1) * L
                    t, pprev = tchunk(c0, pprev)
                    p = pb[bb * RB + rl, pl.ds(c0 - cb, L)]
                    m = t == t
                    d = jnp.where(m, p - t, 0.0)
                    ab2 = ab2 + d * d
                    cb2 = cb2 + jnp.where(m, one, zero)

                def jbody(jj, c2):
                    aa, ca, ab2, cb2, pprev = c2
                    c0 = pl.multiple_of(js * L + 2 * L * jj, L)
                    t, pmid = tchunk(c0, pprev)
                    p = pb[bb * RB + rl, pl.ds(c0 - cb, L)]
                    m = t == t
                    d = jnp.where(m, p - t, 0.0)
                    aa = aa + d * d
                    ca = ca + jnp.where(m, one, zero)
                    t2, pnew = tchunk(c0 + L, pmid)
                    p2 = pb[bb * RB + rl, pl.ds(c0 + L - cb, L)]
                    m2 = t2 == t2
                    d2 = jnp.where(m2, p2 - t2, 0.0)
                    ab2 = ab2 + d2 * d2
                    cb2 = cb2 + jnp.where(m2, one, zero)
                    return aa, ca, ab2, cb2, pnew

                aa, ca, ab2, cb2, _ = lax.fori_loop(
                    0, npair, jbody, (aa, ca, ab2, cb2, pprev))
                return aa, ca, ab2, cb2, seg + (N - rg)

            aa, ca, ab2, cb2, _ = lax.fori_loop(
                0, RB, row, (aa, ca, ab2, cb2, 128 + delta))
            return aa, ca, ab2, cb2

        return lax.fori_loop(0, 2, batch_body, accs)

    z = jnp.zeros((L,), jnp.float32)
    accs = (z, z, z, z)
    hs = [None, None]
    hs[0] = issue(0)
    for k in range(NBLK):
        if k + 1 < NBLK:
            hs[(k + 1) & 1] = issue(k + 1)
        hps, ht = hs[k & 1]
        for h in hps:
            h.wait()
        ht.wait()
        accs = compute(k, accs)

    obuf[pl.ds(0, L)] = accs[0] + accs[2]
    obuf[pl.ds(L, L)] = accs[1] + accs[3]
    pltpu.sync_copy(obuf, out_hbm.at[wid])


@jax.jit
def kernel(input, target):
    inp = input.reshape(B * N, N)
    mesh = plsc.VectorSubcoreMesh(core_axis_name="c", subcore_axis_name="s")
    run = functools.partial(
        pl.kernel,
        mesh=mesh,
        out_type=jax.ShapeDtypeStruct((NW, 2 * L), jnp.float32),
        scratch_types=[
            pltpu.VMEM((2 * RB, N), jnp.float32),
            pltpu.VMEM((2 * RB, N), jnp.float32),
            pltpu.VMEM((8, TROWMAX), jnp.float32),
            pltpu.VMEM((8, TROWMAX), jnp.float32),
            pltpu.VMEM((2 * L,), jnp.float32),
            pltpu.SemaphoreType.DMA,
            pltpu.SemaphoreType.DMA,
            pltpu.SemaphoreType.DMA,
            pltpu.SemaphoreType.DMA,
        ],
    )(_sc_body)
    out = run(inp, target)
    ssum = out[:, :L].sum()
    cnt = out[:, L:].sum()
    return jnp.where(cnt == 0, jnp.float32(0.0),
                     ssum / jnp.maximum(cnt, 1.0))


# speedup vs baseline: 2.2880x; 1.0013x over previous
import functools

import jax
import jax.numpy as jnp
from jax import lax
from jax.experimental import pallas as pl
from jax.experimental.pallas import tpu as pltpu
from jax.experimental.pallas import tpu_sc as plsc

NC, NS, L = 2, 16, 16
NW = NC * NS
B = 64
N = 512
TE = N * (N + 1) // 2
RB = 8
NBLK = N // RB
TROWMAX = 4480


def _blk(k):
    rb = k * RB
    cb = rb & ~127
    W = N - cb
    lb = RB * (N - rb) - 28
    bufw = ((lb + 128 + 127) // 128) * 128
    orb = rb * N - rb * (rb - 1) // 2
    sl = min(orb & ~127, TE - bufw)
    return rb, cb, W, bufw, sl, orb - sl


def _sc_body(inp_hbm, tgt_hbm, out_hbm,
             pb0, pb1, tb0, tb1, obuf, ps0, ps1, ts0, ts1):
    wid = lax.axis_index("s") * NC + lax.axis_index("c")
    g = wid // 4
    ro = (wid % 4) * 2
    b0 = pl.multiple_of(g * 8, 8)
    lanes = lax.iota(jnp.int32, L)
    pbufs, tbufs = (pb0, pb1), (tb0, tb1)
    psems, tsems = (ps0, ps1), (ts0, ts1)

    def issue(k):
        par = k & 1
        rb, cb, W, bufw, sl, _ = _blk(k)
        hps = []
        for bb in range(2):
            r0 = pl.multiple_of((b0 + ro + bb) * N + rb, 8)
            hps.append(pltpu.async_copy(
                inp_hbm.at[pl.ds(r0, RB), pl.ds(cb, W)],
                pbufs[par].at[pl.ds(bb * RB, RB), pl.ds(0, W)],
                psems[par]))
        ht = pltpu.async_copy(
            tgt_hbm.at[pl.ds(b0, 8), pl.ds(sl, bufw)],
            tbufs[par].at[:, pl.ds(128, bufw)], tsems[par])
        return hps, ht

    def compute(k, accs):
        par = k & 1
        pb, tb = pbufs[par], tbufs[par]
        rb, cb, W, bufw, sl, delta = _blk(k)
        jlo = rb // L
        nst = (N // L - 1) - jlo
        js = jlo + 1 + (nst & 1)
        npair = (N // L - js) // 2
        one = jnp.float32(1.0)
        zero = jnp.float32(0.0)

        def batch_body(bb, cab):
            aa, ca, ab2, cb2 = cab
            row2 = ro + bb

            def row(rl, carry):
                aa, ca, ab2, cb2, seg = carry
                rg = rb + rl
                obase = seg - rg
                sh = obase & 15
                ob16 = obase - sh
                pidx = (lanes + sh) & 15
                low = lanes < (16 - sh)

                def tchunk(c0, pb_prev):
                    bv = tb[row2,
                            pl.ds(pl.multiple_of(ob16 + c0 + L, L), L)]
                    pb_new = jnp.take_along_axis(bv, pidx, 0)
                    return jnp.where(low, pb_prev, pb_new), pb_new

                c0 = jlo * L
                av = tb[row2, pl.ds(pl.multiple_of(ob16 + c0, L), L)]
                pa = jnp.take_along_axis(av, pidx, 0)
                t, pprev = tchunk(c0, pa)
                p = pb[bb * RB + rl, pl.ds(c0 - cb, L)]
                m = (lanes >= rg - c0) & (t == t)
                d = jnp.where(m, p - t, 0.0)
                aa = aa + d * d
                ca = ca + jnp.where(m, one, zero)

                if nst & 1:
                    c0 = (jlo + 1) * L
                    t, pprev = tchunk(c0, pprev)
                    p = pb[bb * RB + rl, pl.ds(c0 - cb, L)]
                    m = t == t
                    d = jnp.where(m, p - t, 0.0)
                    ab2 = ab2 + d * d
                    cb2 = cb2 + jnp.where(m, one, zero)

                def jbody(jj, c2):
                    aa, ca, ab2, cb2, pprev = c2
                    c0 = pl.multiple_of(js * L + 2 * L * jj, L)
                    t, pmid = tchunk(c0, pprev)
                    p = pb[bb * RB + rl, pl.ds(c0 - cb, L)]
                    m = t == t
                    d = jnp.where(m, p - t, 0.0)
                    aa = aa + d * d
                    ca = ca + jnp.where(m, one, zero)
                    t2, pnew = tchunk(c0 + L, pmid)
                    p2 = pb[bb * RB + rl, pl.ds(c0 + L - cb, L)]
                    m2 = t2 == t2
                    d2 = jnp.where(m2, p2 - t2, 0.0)
                    ab2 = ab2 + d2 * d2
                    cb2 = cb2 + jnp.where(m2, one, zero)
                    return aa, ca, ab2, cb2, pnew

                aa, ca, ab2, cb2, _ = plsc.parallel_loop(
                    0, npair, 1, unroll=2,
                    carry=(aa, ca, ab2, cb2, pprev))(jbody)
                return aa, ca, ab2, cb2, seg + (N - rg)

            aa, ca, ab2, cb2, _ = lax.fori_loop(
                0, RB, row, (aa, ca, ab2, cb2, 128 + delta))
            return aa, ca, ab2, cb2

        return lax.fori_loop(0, 2, batch_body, accs)

    z = jnp.zeros((L,), jnp.float32)
    accs = (z, z, z, z)
    hs = [None, None]
    hs[0] = issue(0)
    for k in range(NBLK):
        if k + 1 < NBLK:
            hs[(k + 1) & 1] = issue(k + 1)
        hps, ht = hs[k & 1]
        for h in hps:
            h.wait()
        ht.wait()
        accs = compute(k, accs)

    obuf[pl.ds(0, L)] = accs[0] + accs[2]
    obuf[pl.ds(L, L)] = accs[1] + accs[3]
    pltpu.sync_copy(obuf, out_hbm.at[wid])


@jax.jit
def kernel(input, target):
    inp = input.reshape(B * N, N)
    mesh = plsc.VectorSubcoreMesh(core_axis_name="c", subcore_axis_name="s")
    run = functools.partial(
        pl.kernel,
        mesh=mesh,
        out_type=jax.ShapeDtypeStruct((NW, 2 * L), jnp.float32),
        scratch_types=[
            pltpu.VMEM((2 * RB, N), jnp.float32),
            pltpu.VMEM((2 * RB, N), jnp.float32),
            pltpu.VMEM((8, TROWMAX), jnp.float32),
            pltpu.VMEM((8, TROWMAX), jnp.float32),
            pltpu.VMEM((2 * L,), jnp.float32),
            pltpu.SemaphoreType.DMA,
            pltpu.SemaphoreType.DMA,
            pltpu.SemaphoreType.DMA,
            pltpu.SemaphoreType.DMA,
        ],
    )(_sc_body)
    out = run(inp, target)
    ssum = out[:, :L].sum()
    cnt = out[:, L:].sum()
    return jnp.where(cnt == 0, jnp.float32(0.0),
                     ssum / jnp.maximum(cnt, 1.0))


# speedup vs baseline: 2.7278x; 1.1922x over previous
import functools

import jax
import jax.numpy as jnp
from jax import lax
from jax.experimental import pallas as pl
from jax.experimental.pallas import tpu as pltpu
from jax.experimental.pallas import tpu_sc as plsc

NC, NS, L = 2, 16, 16
NW = NC * NS
B = 64
N = 512
TE = N * (N + 1) // 2
RB = 32
NBLK = N // RB
TROWMAX = 4480


def _blk(k):
    rb = k * RB
    cb = rb & ~127
    W = N - cb
    lb = 8 * (N - rb) - 28
    sbw = ((lb + 128 + 127) // 128) * 128
    return rb, cb, W, sbw


def _sc_body(inp_hbm, tgt_hbm, out_hbm,
             pb0, pb1, tb0, tb1, obuf, ps0, ps1, ts0, ts1):
    wid = lax.axis_index("s") * NC + lax.axis_index("c")
    g = wid // 4
    q = wid % 4
    b0 = pl.multiple_of(g * 8, 8)
    lanes = lax.iota(jnp.int32, L)
    pbufs, tbufs = (pb0, pb1), (tb0, tb1)
    psems, tsems = (ps0, ps1), (ts0, ts1)

    def member_base(k):
        rb, cb, W, sbw = _blk(k)
        rq = rb + 8 * q
        orb = rq * N - (rq * (rq - 1)) // 2
        sl = pl.multiple_of(
            jnp.minimum(orb & ~127, TE - sbw), 128)
        return rq, orb, sl

    def issue_slab(k):
        rb, cb, W, sbw = _blk(k)
        _, _, sl = member_base(k)
        return pltpu.async_copy(
            tgt_hbm.at[pl.ds(b0, 8), pl.ds(sl, sbw)],
            tbufs[k & 1].at[:, pl.ds(128, sbw)], tsems[k & 1])

    def issue_pred(gi):
        k, h = divmod(gi, 2)
        rb, cb, W, sbw = _blk(k)
        rq, _, _ = member_base(k)
        hs = []
        for bb in range(4):
            r0 = pl.multiple_of((b0 + h * 4 + bb) * N + rq, 8)
            hs.append(pltpu.async_copy(
                inp_hbm.at[pl.ds(r0, 8), pl.ds(cb, W)],
                pbufs[gi & 1].at[pl.ds(bb * 8, 8), pl.ds(0, W)],
                psems[gi & 1]))
        return hs

    def compute(gi, accs):
        k, h = divmod(gi, 2)
        pb, tb = pbufs[gi & 1], tbufs[k & 1]
        rb, cb, W, sbw = _blk(k)
        rq, orb, sl = member_base(k)
        delta = orb - sl
        jlo = rb // L
        nst = (N // L - 1) - jlo
        js = jlo + 1 + (nst & 1)
        npair = (N // L - js) // 2
        one = jnp.float32(1.0)
        zero = jnp.float32(0.0)

        def batch_body(bb, cab):
            aa, ca, ab2, cb2 = cab
            row2 = h * 4 + bb

            def row(rl, carry):
                aa, ca, ab2, cb2, seg = carry
                rg = rq + rl
                obase = seg - rg
                sh = obase & 15
                ob16 = obase - sh
                pidx = (lanes + sh) & 15
                low = lanes < (16 - sh)

                def tchunk(c0, pb_prev):
                    bv = tb[row2,
                            pl.ds(pl.multiple_of(ob16 + c0 + L, L), L)]
                    pb_new = jnp.take_along_axis(bv, pidx, 0)
                    return jnp.where(low, pb_prev, pb_new), pb_new

                def chunk1(c0, pprev, a, cn):
                    t, pprev = tchunk(c0, pprev)
                    p = pb[bb * 8 + rl, pl.ds(c0 - cb, L)]
                    m = (lanes >= rg - c0) & (t == t)
                    d = jnp.where(m, p - t, 0.0)
                    return pprev, a + d * d, cn + jnp.where(m, one, zero)

                c0 = jlo * L
                av = tb[row2, pl.ds(pl.multiple_of(ob16 + c0, L), L)]
                pprev = jnp.take_along_axis(av, pidx, 0)
                pprev, aa, ca = chunk1(c0, pprev, aa, ca)
                if nst & 1:
                    pprev, ab2, cb2 = chunk1((jlo + 1) * L, pprev, ab2, cb2)

                def jbody(jj, c2):
                    aa, ca, ab2, cb2, pprev = c2
                    c0 = pl.multiple_of(js * L + 2 * L * jj, L)
                    pprev, aa, ca = chunk1(c0, pprev, aa, ca)
                    pprev, ab2, cb2 = chunk1(c0 + L, pprev, ab2, cb2)
                    return aa, ca, ab2, cb2, pprev

                aa, ca, ab2, cb2, _ = lax.fori_loop(
                    0, npair, jbody, (aa, ca, ab2, cb2, pprev))
                return aa, ca, ab2, cb2, seg + (N - rg)

            aa, ca, ab2, cb2, _ = lax.fori_loop(
                0, 8, row, (aa, ca, ab2, cb2, 128 + delta))
            return aa, ca, ab2, cb2

        return lax.fori_loop(0, 4, batch_body, accs)

    z = jnp.zeros((L,), jnp.float32)
    accs = (z, z, z, z)
    hslab = [None, None]
    hpred = [None, None]
    hslab[0] = issue_slab(0)
    hpred[0] = issue_pred(0)
    NGRP = 2 * NBLK
    for gi in range(NGRP):
        k, h = divmod(gi, 2)
        if gi + 1 < NGRP:
            hpred[(gi + 1) & 1] = issue_pred(gi + 1)
        if h == 0 and k + 1 < NBLK:
            hslab[(k + 1) & 1] = issue_slab(k + 1)
        if h == 0:
            hslab[k & 1].wait()
        for hh in hpred[gi & 1]:
            hh.wait()
        accs = compute(gi, accs)

    obuf[pl.ds(0, L)] = accs[0] + accs[2]
    obuf[pl.ds(L, L)] = accs[1] + accs[3]
    pltpu.sync_copy(obuf, out_hbm.at[wid])


@jax.jit
def kernel(input, target):
    inp = input.reshape(B * N, N)
    mesh = plsc.VectorSubcoreMesh(core_axis_name="c", subcore_axis_name="s")
    run = functools.partial(
        pl.kernel,
        mesh=mesh,
        out_type=jax.ShapeDtypeStruct((NW, 2 * L), jnp.float32),
        scratch_types=[
            pltpu.VMEM((4 * 8, N), jnp.float32),
            pltpu.VMEM((4 * 8, N), jnp.float32),
            pltpu.VMEM((8, TROWMAX), jnp.float32),
            pltpu.VMEM((8, TROWMAX), jnp.float32),
            pltpu.VMEM((2 * L,), jnp.float32),
            pltpu.SemaphoreType.DMA,
            pltpu.SemaphoreType.DMA,
            pltpu.SemaphoreType.DMA,
            pltpu.SemaphoreType.DMA,
        ],
    )(_sc_body)
    out = run(inp, target)
    ssum = out[:, :L].sum()
    cnt = out[:, L:].sum()
    return jnp.where(cnt == 0, jnp.float32(0.0),
                     ssum / jnp.maximum(cnt, 1.0))


# speedup vs baseline: 2.7343x; 1.0024x over previous
import functools

import jax
import jax.numpy as jnp
from jax import lax
from jax.experimental import pallas as pl
from jax.experimental.pallas import tpu as pltpu
from jax.experimental.pallas import tpu_sc as plsc

NC, NS, L = 2, 16, 16
NW = NC * NS
B = 64
N = 512
TE = N * (N + 1) // 2
RB = 32
NBLK = N // RB
TROWMAX = 4480


def _blk(k):
    rb = k * RB
    cb = rb & ~127
    W = N - cb
    lb = 8 * (N - rb) - 28
    sbw = ((lb + 128 + 127) // 128) * 128
    return rb, cb, W, sbw


def _sc_body(inp_hbm, tgt_hbm, out_hbm,
             pb0, pb1, tb0, tb1, obuf, ps0, ps1, ts0, ts1):
    wid = lax.axis_index("s") * NC + lax.axis_index("c")
    g = wid // 4
    q = wid % 4
    b0 = pl.multiple_of(g * 8, 8)
    lanes = lax.iota(jnp.int32, L)
    pbufs, tbufs = (pb0, pb1), (tb0, tb1)
    psems, tsems = (ps0, ps1), (ts0, ts1)

    def member_base(k):
        rb, cb, W, sbw = _blk(k)
        rq = rb + 8 * q
        orb = rq * N - (rq * (rq - 1)) // 2
        sl = pl.multiple_of(
            jnp.minimum(orb & ~127, TE - sbw), 128)
        return rq, orb, sl

    def issue_slab(k):
        rb, cb, W, sbw = _blk(k)
        _, _, sl = member_base(k)
        return pltpu.async_copy(
            tgt_hbm.at[pl.ds(b0, 8), pl.ds(sl, sbw)],
            tbufs[k & 1].at[:, pl.ds(128, sbw)], tsems[k & 1])

    def issue_pred(gi):
        k, h = divmod(gi, 2)
        rb, cb, W, sbw = _blk(k)
        rq, _, _ = member_base(k)
        return [pltpu.async_copy(
            inp_hbm.at[pl.ds(b0 + h * 4, 4),
                       pl.ds(pl.multiple_of(rq, 8), 8), pl.ds(cb, W)],
            pbufs[gi & 1].at[:, :, pl.ds(0, W)],
            psems[gi & 1])]

    def compute(gi, accs):
        k, h = divmod(gi, 2)
        pb, tb = pbufs[gi & 1], tbufs[k & 1]
        rb, cb, W, sbw = _blk(k)
        rq, orb, sl = member_base(k)
        delta = orb - sl
        jlo = rb // L
        nst = (N // L - 1) - jlo
        js = jlo + 1 + (nst & 1)
        npair = (N // L - js) // 2
        one = jnp.float32(1.0)
        zero = jnp.float32(0.0)

        def batch_body(bb, cab):
            aa, ca, ab2, cb2 = cab
            row2 = h * 4 + bb

            def row(rl, carry):
                aa, ca, ab2, cb2, seg = carry
                rg = rq + rl
                obase = seg - rg
                sh = obase & 15
                ob16 = obase - sh
                pidx = (lanes + sh) & 15
                low = lanes < (16 - sh)

                def tchunk(c0, pb_prev):
                    bv = tb[row2,
                            pl.ds(pl.multiple_of(ob16 + c0 + L, L), L)]
                    pb_new = jnp.take_along_axis(bv, pidx, 0)
                    return jnp.where(low, pb_prev, pb_new), pb_new

                def chunk1(c0, pprev, a, cn):
                    t, pprev = tchunk(c0, pprev)
                    p = pb[bb, rl, pl.ds(c0 - cb, L)]
                    m = (lanes >= rg - c0) & (t == t)
                    d = jnp.where(m, p - t, 0.0)
                    return pprev, a + d * d, cn + jnp.where(m, one, zero)

                c0 = jlo * L
                av = tb[row2, pl.ds(pl.multiple_of(ob16 + c0, L), L)]
                pprev = jnp.take_along_axis(av, pidx, 0)
                pprev, aa, ca = chunk1(c0, pprev, aa, ca)
                if nst & 1:
                    pprev, ab2, cb2 = chunk1((jlo + 1) * L, pprev, ab2, cb2)

                def jbody(jj, c2):
                    aa, ca, ab2, cb2, pprev = c2
                    c0 = pl.multiple_of(js * L + 2 * L * jj, L)
                    pprev, aa, ca = chunk1(c0, pprev, aa, ca)
                    pprev, ab2, cb2 = chunk1(c0 + L, pprev, ab2, cb2)
                    return aa, ca, ab2, cb2, pprev

                aa, ca, ab2, cb2, _ = lax.fori_loop(
                    0, npair, jbody, (aa, ca, ab2, cb2, pprev))
                return aa, ca, ab2, cb2, seg + (N - rg)

            aa, ca, ab2, cb2, _ = lax.fori_loop(
                0, 8, row, (aa, ca, ab2, cb2, 128 + delta))
            return aa, ca, ab2, cb2

        return lax.fori_loop(0, 4, batch_body, accs)

    z = jnp.zeros((L,), jnp.float32)
    accs = (z, z, z, z)
    hslab = [None, None]
    hpred = [None, None]
    hslab[0] = issue_slab(0)
    hpred[0] = issue_pred(0)
    NGRP = 2 * NBLK
    for gi in range(NGRP):
        k, h = divmod(gi, 2)
        if gi + 1 < NGRP:
            hpred[(gi + 1) & 1] = issue_pred(gi + 1)
        if h == 0 and k + 1 < NBLK:
            hslab[(k + 1) & 1] = issue_slab(k + 1)
        if h == 0:
            hslab[k & 1].wait()
        for hh in hpred[gi & 1]:
            hh.wait()
        accs = compute(gi, accs)

    obuf[pl.ds(0, L)] = accs[0] + accs[2]
    obuf[pl.ds(L, L)] = accs[1] + accs[3]
    pltpu.sync_copy(obuf, out_hbm.at[wid])


@jax.jit
def kernel(input, target):
    inp = input.reshape(B, N, N)
    mesh = plsc.VectorSubcoreMesh(core_axis_name="c", subcore_axis_name="s")
    run = functools.partial(
        pl.kernel,
        mesh=mesh,
        out_type=jax.ShapeDtypeStruct((NW, 2 * L), jnp.float32),
        scratch_types=[
            pltpu.VMEM((4, 8, N), jnp.float32),
            pltpu.VMEM((4, 8, N), jnp.float32),
            pltpu.VMEM((8, TROWMAX), jnp.float32),
            pltpu.VMEM((8, TROWMAX), jnp.float32),
            pltpu.VMEM((2 * L,), jnp.float32),
            pltpu.SemaphoreType.DMA,
            pltpu.SemaphoreType.DMA,
            pltpu.SemaphoreType.DMA,
            pltpu.SemaphoreType.DMA,
        ],
    )(_sc_body)
    out = run(inp, target)
    ssum = out[:, :L].sum()
    cnt = out[:, L:].sum()
    return jnp.where(cnt == 0, jnp.float32(0.0),
                     ssum / jnp.maximum(cnt, 1.0))
